# Initial kernel scaffold; baseline (speedup 1.0000x reference)
import functools

import jax
import jax.numpy as jnp
from jax import lax
from jax.experimental import pallas as pl
from jax.experimental.pallas import tpu as pltpu
from jax.experimental.pallas import tpu_sc as plsc

F = 100
C = 1000
D = 32
B = 16384
R = B * F

LANES = 16
DMA_N = 128
N_DMA = 8
CHUNK = DMA_N * N_DMA


def _make_kernel():
  mesh = plsc.VectorSubcoreMesh(core_axis_name="c", subcore_axis_name="s")
  nw = mesh.num_cores * mesh.num_subcores
  per_w = R // nw
  n_chunk = per_w // CHUNK
  assert per_w % CHUNK == 0

  def body(x_hbm, tbl_hbm, out_hbm, idx_v, rows_v, sem):
    wid = lax.axis_index("s") * mesh.num_cores + lax.axis_index("c")
    row0_w = wid * (per_w // DMA_N)

    @pl.loop(0, n_chunk)
    def chunk_loop(ci):
      row0 = row0_w + ci * N_DMA
      base = row0 * DMA_N
      pltpu.sync_copy(x_hbm.at[pl.ds(row0, N_DMA)], idx_v)
      for r in range(N_DMA):
        for c16 in range(DMA_N // LANES):
          pos = (base + r * DMA_N + c16 * LANES) + lax.iota(jnp.int32, (LANES,))
          fld = lax.rem(pos, F)
          raw = idx_v[r, pl.ds(c16 * LANES, LANES)]
          idx_v[r, pl.ds(c16 * LANES, LANES)] = raw + fld * C
      copies = [
          pltpu.async_copy(
              tbl_hbm.at[idx_v.at[j]],
              rows_v.at[pl.ds(j * DMA_N, DMA_N), :],
              sem,
          )
          for j in range(N_DMA)
      ]
      for cp in copies:
        cp.wait()
      pltpu.sync_copy(rows_v, out_hbm.at[pl.ds(base, CHUNK), :])

  return pl.kernel(
      body,
      out_type=jax.ShapeDtypeStruct((R, D), jnp.float32),
      mesh=mesh,
      scratch_types=[
          pltpu.VMEM((N_DMA, DMA_N), jnp.int32),
          pltpu.VMEM((CHUNK, D), jnp.float32),
          pltpu.SemaphoreType.DMA,
      ],
  )


def kernel(x, weight):
  x2 = x.astype(jnp.int32).reshape(R // DMA_N, DMA_N)
  tbl = weight.reshape(F * C, D)
  out = _make_kernel()(x2, tbl)
  return out.reshape(B, F, D)


# speedup vs baseline: 5.1643x; 5.1643x over previous
import functools

import jax
import jax.numpy as jnp
from jax import lax
from jax.experimental import pallas as pl
from jax.experimental.pallas import tpu as pltpu
from jax.experimental.pallas import tpu_sc as plsc

F = 100
C = 1000
D = 32
B = 16384
R = B * F

LANES = 16
DMA_N = 128
N_DMA = 8
CHUNK = DMA_N * N_DMA


def _make_kernel():
  mesh = plsc.VectorSubcoreMesh(core_axis_name="c", subcore_axis_name="s")
  nw = mesh.num_cores * mesh.num_subcores
  per_w = R // nw
  n_chunk = per_w // CHUNK
  assert per_w % CHUNK == 0

  def body(x_hbm, tbl_hbm, out_hbm, idx_v, rows_v, sem):
    wid = lax.axis_index("s") * mesh.num_cores + lax.axis_index("c")
    row0_w = wid * (per_w // DMA_N)

    @pl.loop(0, n_chunk)
    def chunk_loop(ci):
      row0 = row0_w + ci * N_DMA
      base = row0 * DMA_N
      pltpu.sync_copy(x_hbm.at[pl.ds(row0, N_DMA)], idx_v)
      for r in range(N_DMA):
        for c16 in range(DMA_N // LANES):
          pos = (base + r * DMA_N + c16 * LANES) + lax.iota(jnp.int32, LANES)
          fld = lax.rem(pos, F)
          raw = idx_v[r, pl.ds(c16 * LANES, LANES)]
          idx_v[r, pl.ds(c16 * LANES, LANES)] = raw + fld * C
      copies = [
          pltpu.async_copy(
              tbl_hbm.at[idx_v.at[j]],
              rows_v.at[pl.ds(j * DMA_N, DMA_N), :],
              sem,
          )
          for j in range(N_DMA)
      ]
      for cp in copies:
        cp.wait()
      pltpu.sync_copy(rows_v, out_hbm.at[pl.ds(base, CHUNK), :])

  return pl.kernel(
      body,
      out_type=jax.ShapeDtypeStruct((R, D), jnp.float32),
      mesh=mesh,
      scratch_types=[
          pltpu.VMEM((N_DMA, DMA_N), jnp.int32),
          pltpu.VMEM((CHUNK, D), jnp.float32),
          pltpu.SemaphoreType.DMA,
      ],
      compiler_params=pltpu.CompilerParams(use_tc_tiling_on_sc=False),
  )


def kernel(x, weight):
  x2 = x.astype(jnp.int32).reshape(R // DMA_N, DMA_N)
  tbl = weight.reshape(F * C, D)
  out = _make_kernel()(x2, tbl)
  return out.reshape(B, F, D)


# speedup vs baseline: 21.3099x; 4.1264x over previous
import functools

import jax
import jax.numpy as jnp
from jax import lax
from jax.experimental import pallas as pl
from jax.experimental.pallas import tpu as pltpu
from jax.experimental.pallas import tpu_sc as plsc

F = 100
C = 1000
D = 32
B = 16384
LANES = 16

BT = 128
N_BT = B // BT
G = 4
GB = G * BT


def _make_kernel():
  mesh = plsc.VectorSubcoreMesh(core_axis_name="c", subcore_axis_name="s")
  nw = mesh.num_cores * mesh.num_subcores
  assert N_BT == nw * G

  def body(xt_hbm, tbl_hbm, out_hbm, idx_all, rows0, rows1, trans, gsem0, gsem1):
    wid = lax.axis_index("s") * mesh.num_cores + lax.axis_index("c")
    ri = lax.iota(jnp.int32, LANES)

    pltpu.sync_copy(xt_hbm.at[:, pl.ds(wid * G, G), :], idx_all)

    @pl.loop(0, F)
    def adjust(f):
      off = f * C
      for j in range(G):
        for i in range(BT // LANES):
          sl = idx_all.at[f, j, pl.ds(i * LANES, LANES)]
          idx_all[f, j, pl.ds(i * LANES, LANES)] = sl[...] + off

    def fire_gathers(f, rows, gsem):
      return [
          pltpu.async_copy(
              tbl_hbm.at[idx_all.at[f, j]],
              rows.at[pl.ds(j * BT, BT), :],
              gsem,
          )
          for j in range(G)
      ]

    def drain(rows, gsem):
      for j in range(G):
        pltpu.make_async_copy(
            tbl_hbm.at[idx_all.at[0, j]], rows.at[pl.ds(j * BT, BT), :], gsem
        ).wait()

    def process(f, rows):
      for dt in range(D // 8):
        for bt in range(G):
          for s in range(8):
            d = dt * 8 + s
            for i in range(BT // LANES):
              blk = rows.at[pl.ds(bt * BT + i * LANES, LANES), :]
              v = plsc.load_gather(blk, [ri, jnp.full((LANES,), d, jnp.int32)])
              trans[dt, bt, s, pl.ds(i * LANES, LANES)] = v
      for dt in range(D // 8):
        pltpu.sync_copy(trans.at[dt], out_hbm.at[f, dt, pl.ds(wid * G, G)])

    fire_gathers(0, rows0, gsem0)

    @pl.loop(0, F, step=2)
    def floop(f):
      fire_gathers(f + 1, rows1, gsem1)
      drain(rows0, gsem0)
      process(f, rows0)

      @pl.when(f + 2 < F)
      def _():
        fire_gathers(f + 2, rows0, gsem0)

      drain(rows1, gsem1)
      process(f + 1, rows1)

  return pl.kernel(
      body,
      out_type=jax.ShapeDtypeStruct((F, D // 8, N_BT, 8, BT), jnp.float32),
      mesh=mesh,
      scratch_types=[
          pltpu.VMEM((F, G, BT), jnp.int32),
          pltpu.VMEM((GB, D), jnp.float32),
          pltpu.VMEM((GB, D), jnp.float32),
          pltpu.VMEM((D // 8, G, 8, BT), jnp.float32),
          pltpu.SemaphoreType.DMA,
          pltpu.SemaphoreType.DMA,
      ],
      compiler_params=pltpu.CompilerParams(
          use_tc_tiling_on_sc=False, needs_layout_passes=False
      ),
  )


def kernel(x, weight):
  xt = x.astype(jnp.int32).T.reshape(F, N_BT, BT)
  tbl = weight.reshape(F * C, D)
  u = _make_kernel()(xt, tbl)
  return jnp.transpose(u, (2, 4, 0, 1, 3)).reshape(B, F, D)


# speedup vs baseline: 22.5870x; 1.0599x over previous
import functools

import jax
import jax.numpy as jnp
from jax import lax
from jax.experimental import pallas as pl
from jax.experimental.pallas import tpu as pltpu
from jax.experimental.pallas import tpu_sc as plsc

F = 100
C = 1000
D = 32
B = 16384
LANES = 16

BT = 128
N_BT = B // BT
G = 4
GB = G * BT


def _make_kernel():
  mesh = plsc.VectorSubcoreMesh(core_axis_name="c", subcore_axis_name="s")
  nw = mesh.num_cores * mesh.num_subcores
  assert N_BT == nw * G

  def body(xt_hbm, tbl_hbm, out_hbm, idx_all, rows0, rows1, trans0, trans1,
           gsem0, gsem1, osem0, osem1):
    wid = lax.axis_index("s") * mesh.num_cores + lax.axis_index("c")
    ri = lax.iota(jnp.int32, LANES)
    dvecs = [jnp.full((LANES,), d, jnp.int32) for d in range(D)]

    pltpu.sync_copy(xt_hbm.at[:, pl.ds(wid * G, G), :], idx_all)

    @pl.loop(0, F)
    def adjust(f):
      off = f * C
      for j in range(G):
        for i in range(BT // LANES):
          sl = idx_all.at[f, j, pl.ds(i * LANES, LANES)]
          idx_all[f, j, pl.ds(i * LANES, LANES)] = sl[...] + off

    def fire_gathers(f, rows, gsem):
      for j in range(G):
        pltpu.async_copy(
            tbl_hbm.at[idx_all.at[f, j]],
            rows.at[pl.ds(j * BT, BT), :],
            gsem,
        )

    def drain_gathers(rows, gsem):
      for j in range(G):
        pltpu.make_async_copy(
            tbl_hbm.at[idx_all.at[0, j]], rows.at[pl.ds(j * BT, BT), :], gsem
        ).wait()

    def transpose(rows, trans):
      for dt in range(D // 8):
        for s in range(8):
          dv = dvecs[dt * 8 + s]
          for bt in range(G):
            for i in range(BT // LANES):
              blk = rows.at[pl.ds(bt * BT + i * LANES, LANES), :]
              trans[dt, bt, s, pl.ds(i * LANES, LANES)] = plsc.load_gather(
                  blk, [ri, dv])

    def fire_out(f, trans, osem):
      for dt in range(D // 8):
        pltpu.async_copy(trans.at[dt], out_hbm.at[f, dt, pl.ds(wid * G, G)],
                         osem)

    def drain_out(trans, osem):
      for dt in range(D // 8):
        pltpu.make_async_copy(
            trans.at[dt], out_hbm.at[0, dt, pl.ds(wid * G, G)], osem
        ).wait()

    fire_gathers(0, rows0, gsem0)

    @pl.loop(0, F, step=2)
    def floop(f):
      fire_gathers(f + 1, rows1, gsem1)
      drain_gathers(rows0, gsem0)

      @pl.when(f > 0)
      def _():
        drain_out(trans0, osem0)

      transpose(rows0, trans0)
      fire_out(f, trans0, osem0)

      @pl.when(f + 2 < F)
      def _():
        fire_gathers(f + 2, rows0, gsem0)

      drain_gathers(rows1, gsem1)

      @pl.when(f > 0)
      def _():
        drain_out(trans1, osem1)

      transpose(rows1, trans1)
      fire_out(f + 1, trans1, osem1)

    drain_out(trans0, osem0)
    drain_out(trans1, osem1)

  return pl.kernel(
      body,
      out_type=jax.ShapeDtypeStruct((F, D // 8, N_BT, 8, BT), jnp.float32),
      mesh=mesh,
      scratch_types=[
          pltpu.VMEM((F, G, BT), jnp.int32),
          pltpu.VMEM((GB, D), jnp.float32),
          pltpu.VMEM((GB, D), jnp.float32),
          pltpu.VMEM((D // 8, G, 8, BT), jnp.float32),
          pltpu.VMEM((D // 8, G, 8, BT), jnp.float32),
          pltpu.SemaphoreType.DMA,
          pltpu.SemaphoreType.DMA,
          pltpu.SemaphoreType.DMA,
          pltpu.SemaphoreType.DMA,
      ],
      compiler_params=pltpu.CompilerParams(
          use_tc_tiling_on_sc=False, needs_layout_passes=False
      ),
  )


def kernel(x, weight):
  xt = x.astype(jnp.int32).T.reshape(F, N_BT, BT)
  tbl = weight.reshape(F * C, D)
  u = _make_kernel()(xt, tbl)
  return jnp.transpose(u, (2, 4, 0, 1, 3)).reshape(B, F, D)
